# baseline (speedup 1.0000x reference)
import functools

import jax
import jax.numpy as jnp
from jax import lax
from jax.experimental import pallas as pl
from jax.experimental.pallas import tpu as pltpu
from jax.experimental.pallas import tpu_sc as plsc

_LANES = 16
_TR = 8
_C = 16
_NBUF = 4


@functools.cache
def _build(B, V, D):
    info = plsc.get_sparse_core_info()
    nw = info.num_cores * info.num_subcores
    bpw = B // nw
    n_chunks = bpw // _C
    n_blocks = n_chunks // _NBUF
    otiles = _C // _TR
    mesh = plsc.VectorSubcoreMesh(core_axis_name="c", subcore_axis_name="s")

    @functools.partial(
        pl.kernel,
        mesh=mesh,
        out_type=jax.ShapeDtypeStruct((B, D), jnp.float32),
        scratch_types=[
            pltpu.VMEM((bpw,), jnp.int32),
            pltpu.VMEM((bpw,), jnp.int32),
            pltpu.VMEM((bpw,), jnp.float32),
            [pltpu.VMEM((_C, _TR, D), jnp.float32) for _ in range(_NBUF)],
            [pltpu.VMEM((otiles, _TR, D), jnp.float32) for _ in range(2)],
            [pltpu.SemaphoreType.DMA for _ in range(_NBUF)],
            pltpu.SemaphoreType.DMA,
        ],
    )
    def gather_scale(x_hbm, tidx_hbm, ridx_hbm, sc_hbm, out_hbm,
                     tidx_v, ridx_v, sc_v, lands, obufs, sems, osem):
        wid = lax.axis_index("s") * info.num_cores + lax.axis_index("c")
        base = wid * bpw
        obase = wid * (bpw // _TR)
        xv = x_hbm.reshape(V // _TR, _TR, D)
        ov = out_hbm.reshape(B // _TR, _TR, D)
        pltpu.sync_copy(tidx_hbm.at[pl.ds(base, bpw)], tidx_v)
        pltpu.sync_copy(ridx_hbm.at[pl.ds(base, bpw)], ridx_v)
        pltpu.sync_copy(sc_hbm.at[pl.ds(base, bpw)], sc_v)

        def fire(chunk, land, sem):
            t16 = tidx_v[pl.ds(chunk * _C, _LANES)]
            for r in range(_LANES):
                pltpu.async_copy(xv.at[t16[r]], land.at[r], sem)

        def drain(land, sem):
            pltpu.make_async_copy(xv.at[pl.ds(0, _C)], land, sem).wait()

        def drain_out(obuf):
            pltpu.make_async_copy(obuf, ov.at[pl.ds(0, otiles)], osem).wait()

        def process(chunk, land, obuf):
            s16 = sc_v[pl.ds(chunk * _C, _LANES)]
            r16 = ridx_v[pl.ds(chunk * _C, _LANES)]
            for r in range(_LANES):
                rsel = r16[r]
                s = s16[r]
                for j in range(D // _LANES):
                    col = pl.ds(j * _LANES, _LANES)
                    obuf[r // _TR, r % _TR, col] = land[r, rsel, col] * s
            pltpu.async_copy(
                obuf, ov.at[pl.ds(obase + chunk * otiles, otiles)], osem)

        for b in range(_NBUF):
            fire(b, lands[b], sems[b])

        def block(blk, carry):
            for b in range(_NBUF):
                chunk = blk * _NBUF + b
                drain(lands[b], sems[b])
                if b >= 2:
                    drain_out(obufs[b % 2])
                else:
                    @pl.when(blk > 0)
                    def _():
                        drain_out(obufs[b % 2])

                process(chunk, lands[b], obufs[b % 2])

                @pl.when(blk < n_blocks - 1)
                def _():
                    fire(chunk + _NBUF, lands[b], sems[b])
            return carry

        lax.fori_loop(0, n_blocks, block, 0)
        drain_out(obufs[0])
        drain_out(obufs[1])

    def run(X, tidx, ridx, s):
        return gather_scale(X, tidx, ridx, s)

    return run


def kernel(label_idc, scores, X):
    B = label_idc.shape[0]
    V, D = X.shape
    idx = label_idc.astype(jnp.int32)
    tidx = lax.shift_right_logical(idx, 3)
    ridx = lax.bitwise_and(idx, 7)
    s = scores.reshape(B).astype(jnp.float32)
    return _build(B, V, D)(X, tidx, ridx, s)
